# baseline (speedup 1.0000x reference)
import functools

import jax
import jax.numpy as jnp
from jax import lax
from jax.experimental import pallas as pl
from jax.experimental.pallas import tpu as pltpu
from jax.experimental.pallas import tpu_sc as plsc

N_NODES = 10000
N_PAD = 10112
E = 320000
IN_CH = 128
H = 32
NC, NS = 2, 16
NW = NC * NS
GROUP = 256
GPT = 40
E_PAD = NW * GPT * GROUP
ROWS_PT = N_PAD // NS


NBUF = 4
G0, G1 = 40, 40
D0, D1 = 40, 40
GSUM = G0 + G1


def _zero_acc(zbuf, z_acc, rows):
    zv = jnp.zeros((16,), jnp.float32)

    def zrow(r, carry):
        zbuf[r, pl.ds(0, 16)] = zv
        zbuf[r, pl.ds(16, 16)] = zv
        return carry

    lax.fori_loop(0, ROWS_PT, zrow, 0)
    pltpu.sync_copy(zbuf, z_acc.at[rows])


def _pipe(y_hbm, z_acc, src_v, dst_v, rows4, gsem, ssem, G):
    NP = G // NBUF
    for b in range(NBUF):
        pltpu.async_copy(y_hbm.at[src_v.at[b]], rows4.at[b], gsem.at[b])

    def step(P, carry):
        for b in range(NBUF):
            g = P * NBUF + b
            pltpu.make_async_copy(y_hbm.at[src_v.at[g]], rows4.at[b],
                                  gsem.at[b]).wait()
            pltpu.async_copy(rows4.at[b], z_acc.at[dst_v.at[g]], ssem.at[b],
                             add=True)

            @pl.when(P < NP - 1)
            def _refill():
                pltpu.make_async_copy(rows4.at[b], z_acc.at[dst_v.at[g]],
                                      ssem.at[b]).wait()
                pltpu.async_copy(y_hbm.at[src_v.at[g + NBUF]], rows4.at[b],
                                 gsem.at[b])
        return carry

    lax.fori_loop(0, NP, step, 0)
    for b in range(NBUF):
        g = (NP - 1) * NBUF + b
        pltpu.make_async_copy(rows4.at[b], z_acc.at[dst_v.at[g]],
                              ssem.at[b]).wait()


def _edge_pass_body(src_hbm, dst_hbm, y_hbm, z_out,
                    src_v, dst_v, rows4, zbuf, z_acc, gsem, ssem):
    cid = lax.axis_index("c")
    sid = lax.axis_index("s")
    rows = pl.ds(sid * ROWS_PT, ROWS_PT)
    _zero_acc(zbuf, z_acc, rows)

    @pl.when(cid == 0)
    def _stage0():
        base = sid * GSUM
        pltpu.sync_copy(src_hbm.at[pl.ds(base, G0)], src_v.at[pl.ds(0, G0)])
        pltpu.sync_copy(dst_hbm.at[pl.ds(base, G0)], dst_v.at[pl.ds(0, G0)])

    @pl.when(cid == 1)
    def _stage1():
        base = sid * GSUM + G0
        pltpu.sync_copy(src_hbm.at[pl.ds(base, G1)], src_v.at[pl.ds(0, G1)])
        pltpu.sync_copy(dst_hbm.at[pl.ds(base, G1)], dst_v.at[pl.ds(0, G1)])

    plsc.subcore_barrier()

    @pl.when(cid == 0)
    def _run0():
        _pipe(y_hbm, z_acc, src_v, dst_v, rows4, gsem, ssem, G0)

    @pl.when(cid == 1)
    def _run1():
        _pipe(y_hbm, z_acc, src_v, dst_v, rows4, gsem, ssem, G1)

    plsc.subcore_barrier()
    pltpu.sync_copy(z_acc.at[rows], z_out.at[cid].at[rows])


def _deg_fire(z_acc, dst_v, rows_v, ssem, G):
    AHEAD = 8

    def fire(g, carry):
        pltpu.async_copy(rows_v, z_acc.at[dst_v.at[g]], ssem, add=True)

        @pl.when(g >= AHEAD)
        def _drain():
            pltpu.make_async_copy(rows_v, z_acc.at[dst_v.at[g]], ssem).wait()
        return carry

    lax.fori_loop(0, G, fire, 0)

    def drain(g, carry):
        pltpu.make_async_copy(rows_v, z_acc.at[dst_v.at[0]], ssem).wait()
        return carry

    lax.fori_loop(0, AHEAD, drain, 0)


def _deg_pass_body(dst_hbm, z_out, dst_v, rows_v, zbuf, z_acc, ssem):
    cid = lax.axis_index("c")
    sid = lax.axis_index("s")
    rows = pl.ds(sid * ROWS_PT, ROWS_PT)
    _zero_acc(zbuf, z_acc, rows)

    @pl.when(cid == 0)
    def _stage0():
        pltpu.sync_copy(dst_hbm.at[pl.ds(sid * GSUM, D0)],
                        dst_v.at[pl.ds(0, D0)])

    @pl.when(cid == 1)
    def _stage1():
        pltpu.sync_copy(dst_hbm.at[pl.ds(sid * GSUM + D0, D1)],
                        dst_v.at[pl.ds(0, D1)])

    one = jnp.where(lax.iota(jnp.int32, 16) == 0, 1.0, 0.0).astype(jnp.float32)
    zv = jnp.zeros((16,), jnp.float32)

    def fill(r, carry):
        rows_v[r, pl.ds(0, 16)] = one
        rows_v[r, pl.ds(16, 16)] = zv
        return carry

    lax.fori_loop(0, GROUP, fill, 0)
    plsc.subcore_barrier()

    @pl.when(cid == 0)
    def _run0():
        _deg_fire(z_acc, dst_v, rows_v, ssem, D0)

    @pl.when(cid == 1)
    def _run1():
        _deg_fire(z_acc, dst_v, rows_v, ssem, D1)

    plsc.subcore_barrier()
    pltpu.sync_copy(z_acc.at[rows], z_out.at[cid].at[rows])


@functools.cache
def _edge_pass():
    return pl.kernel(
        _edge_pass_body,
        out_type=jax.ShapeDtypeStruct((NC, N_PAD, H), jnp.float32),
        mesh=plsc.VectorSubcoreMesh(core_axis_name="c", subcore_axis_name="s",
                                    num_cores=NC, num_subcores=NS),
        scratch_types=[
            pltpu.VMEM((G0, GROUP), jnp.int32),
            pltpu.VMEM((G0, GROUP), jnp.int32),
            pltpu.VMEM((NBUF, GROUP, H), jnp.float32),
            pltpu.VMEM((ROWS_PT, H), jnp.float32),
            pltpu.VMEM_SHARED((N_PAD, H), jnp.float32),
            pltpu.SemaphoreType.DMA((NBUF,)),
            pltpu.SemaphoreType.DMA((NBUF,)),
        ],
        compiler_params=pltpu.CompilerParams(use_tc_tiling_on_sc=False),
    )


@functools.cache
def _deg_pass():
    return pl.kernel(
        _deg_pass_body,
        out_type=jax.ShapeDtypeStruct((NC, N_PAD, H), jnp.float32),
        mesh=plsc.VectorSubcoreMesh(core_axis_name="c", subcore_axis_name="s",
                                    num_cores=NC, num_subcores=NS),
        scratch_types=[
            pltpu.VMEM((D0, GROUP), jnp.int32),
            pltpu.VMEM((GROUP, H), jnp.float32),
            pltpu.VMEM((ROWS_PT, H), jnp.float32),
            pltpu.VMEM_SHARED((N_PAD, H), jnp.float32),
            pltpu.SemaphoreType.DMA,
        ],
        compiler_params=pltpu.CompilerParams(use_tc_tiling_on_sc=False),
    )


_PREC = jax.lax.Precision.HIGHEST


def _tc1_body(x_ref, w1_ref, degp_ref, xw_ref, y_ref, dinv_ref):
    deg = (jnp.sum(degp_ref[...], axis=(0, 2))[:N_NODES] + 1.0)
    dinv = lax.rsqrt(deg)
    xw = jnp.dot(x_ref[...], w1_ref[...], precision=_PREC,
                 preferred_element_type=jnp.float32)
    xw_ref[...] = xw
    y_ref[...] = jnp.zeros((N_PAD, H), jnp.float32)
    y_ref[pl.ds(0, N_NODES)] = xw * dinv[:, None]
    dinv_ref[...] = dinv


def _tc2_body(zp_ref, xw_ref, dinv_ref, b_ref, w_ref, xw2_ref, y2_ref):
    z = zp_ref[0, :N_NODES, :] + zp_ref[1, :N_NODES, :]
    dinv = dinv_ref[...]
    h = jnp.maximum(dinv[:, None] * z + (dinv * dinv)[:, None] * xw_ref[...]
                    + b_ref[...][None, :], 0.0)
    xw2 = jnp.dot(h, w_ref[...], precision=_PREC,
                  preferred_element_type=jnp.float32)
    xw2_ref[...] = xw2
    y2_ref[...] = jnp.zeros((N_PAD, H), jnp.float32)
    y2_ref[pl.ds(0, N_NODES)] = xw2 * dinv[:, None]


def _tc3_body(zp_ref, xw_ref, dinv_ref, b_ref, wh_ref, bh_ref, cash_ref,
              w0_ref, wr_ref):
    z = zp_ref[0, :N_NODES, :] + zp_ref[1, :N_NODES, :]
    dinv = dinv_ref[...]
    h = jnp.maximum(dinv[:, None] * z + (dinv * dinv)[:, None] * xw_ref[...]
                    + b_ref[...][None, :], 0.0)
    s = jnp.dot(h, wh_ref[...], precision=_PREC,
                preferred_element_type=jnp.float32)[:, 0] + bh_ref[...]
    m = jnp.maximum(jnp.max(s), jnp.max(cash_ref[...]))
    es = jnp.exp(s - m)
    ec = jnp.exp(cash_ref[...] - m)
    tot = jnp.sum(es) + jnp.sum(ec)
    w0_ref[...] = ec / tot
    wr_ref[...] = es / tot


def kernel(x, edge_index, W1, b1, W2, b2, Wh, bh, cash):
    ei = edge_index.astype(jnp.int32)
    pad = E_PAD - E
    src = jnp.concatenate([ei[0], jnp.zeros((pad,), jnp.int32)])
    dst = jnp.concatenate([ei[1], jnp.full((pad,), N_NODES, jnp.int32)])
    src = src.reshape(NW * GPT, GROUP)
    dst = dst.reshape(NW * GPT, GROUP)
    degp = _deg_pass()(dst)
    xw1, y1, dinv = pl.pallas_call(
        _tc1_body,
        out_shape=[
            jax.ShapeDtypeStruct((N_NODES, H), jnp.float32),
            jax.ShapeDtypeStruct((N_PAD, H), jnp.float32),
            jax.ShapeDtypeStruct((N_NODES,), jnp.float32),
        ],
    )(x, W1, degp)
    z1p = _edge_pass()(src, dst, y1)
    xw2, y2 = pl.pallas_call(
        _tc2_body,
        out_shape=[
            jax.ShapeDtypeStruct((N_NODES, H), jnp.float32),
            jax.ShapeDtypeStruct((N_PAD, H), jnp.float32),
        ],
    )(z1p, xw1, dinv, b1, W2)
    z2p = _edge_pass()(src, dst, y2)
    w0, wr = pl.pallas_call(
        _tc3_body,
        out_shape=[
            jax.ShapeDtypeStruct((1,), jnp.float32),
            jax.ShapeDtypeStruct((N_NODES,), jnp.float32),
        ],
    )(z2p, xw2, dinv, b2, Wh, bh, cash)
    return jnp.concatenate([w0, wr], axis=0)


# speedup vs baseline: 1.5852x; 1.5852x over previous
import functools

import jax
import jax.numpy as jnp
from jax import lax
from jax.experimental import pallas as pl
from jax.experimental.pallas import tpu as pltpu
from jax.experimental.pallas import tpu_sc as plsc

N_NODES = 10000
N_PAD = 10112
E = 320000
IN_CH = 128
H = 32
NC, NS = 2, 16
NW = NC * NS
GROUP = 256
GPT = 40
E_PAD = NW * GPT * GROUP
ROWS_PT = N_PAD // NS


NBUF = 4
G0, G1 = 40, 40
D0, D1 = 40, 40
GSUM = G0 + G1


def _zero_acc(zbuf, z_acc, rows):
    zv = jnp.zeros((16,), jnp.float32)

    def zrow(r, carry):
        zbuf[r, pl.ds(0, 16)] = zv
        zbuf[r, pl.ds(16, 16)] = zv
        return carry

    lax.fori_loop(0, ROWS_PT, zrow, 0)
    pltpu.sync_copy(zbuf, z_acc.at[rows])


def _pipe(y_sp, z_acc, src_v, dst_v, rows4, gsem, ssem, G):
    NP = G // NBUF
    for b in range(NBUF):
        pltpu.async_copy(y_sp.at[src_v.at[b]], rows4.at[b], gsem.at[b])

    def step(P, carry):
        for b in range(NBUF):
            g = P * NBUF + b
            pltpu.make_async_copy(y_sp.at[src_v.at[g]], rows4.at[b],
                                  gsem.at[b]).wait()
            pltpu.async_copy(rows4.at[b], z_acc.at[dst_v.at[g]], ssem.at[b],
                             add=True)

            @pl.when(P < NP - 1)
            def _refill():
                pltpu.make_async_copy(rows4.at[b], z_acc.at[dst_v.at[g]],
                                      ssem.at[b]).wait()
                pltpu.async_copy(y_sp.at[src_v.at[g + NBUF]], rows4.at[b],
                                 gsem.at[b])
        return carry

    lax.fori_loop(0, NP, step, 0)
    for b in range(NBUF):
        g = (NP - 1) * NBUF + b
        pltpu.make_async_copy(rows4.at[b], z_acc.at[dst_v.at[g]],
                              ssem.at[b]).wait()


def _edge_pass_body(src_hbm, dst_hbm, y_hbm, z_out,
                    src_v, dst_v, rows4, zbuf, y_sp, z_acc, gsem, ssem):
    cid = lax.axis_index("c")
    sid = lax.axis_index("s")
    rows = pl.ds(sid * ROWS_PT, ROWS_PT)
    _zero_acc(zbuf, z_acc, rows)
    pltpu.sync_copy(y_hbm.at[rows], zbuf)
    pltpu.sync_copy(zbuf, y_sp.at[rows])

    @pl.when(cid == 0)
    def _stage0():
        base = sid * GSUM
        pltpu.sync_copy(src_hbm.at[pl.ds(base, G0)], src_v.at[pl.ds(0, G0)])
        pltpu.sync_copy(dst_hbm.at[pl.ds(base, G0)], dst_v.at[pl.ds(0, G0)])

    @pl.when(cid == 1)
    def _stage1():
        base = sid * GSUM + G0
        pltpu.sync_copy(src_hbm.at[pl.ds(base, G1)], src_v.at[pl.ds(0, G1)])
        pltpu.sync_copy(dst_hbm.at[pl.ds(base, G1)], dst_v.at[pl.ds(0, G1)])

    plsc.subcore_barrier()

    @pl.when(cid == 0)
    def _run0():
        _pipe(y_sp, z_acc, src_v, dst_v, rows4, gsem, ssem, G0)

    @pl.when(cid == 1)
    def _run1():
        _pipe(y_sp, z_acc, src_v, dst_v, rows4, gsem, ssem, G1)

    plsc.subcore_barrier()
    pltpu.sync_copy(z_acc.at[rows], z_out.at[cid].at[rows])


def _deg_fire(z_acc, dst_v, rows_v, ssem, G):
    AHEAD = 8

    def fire(g, carry):
        pltpu.async_copy(rows_v, z_acc.at[dst_v.at[g]], ssem, add=True)

        @pl.when(g >= AHEAD)
        def _drain():
            pltpu.make_async_copy(rows_v, z_acc.at[dst_v.at[g]], ssem).wait()
        return carry

    lax.fori_loop(0, G, fire, 0)

    def drain(g, carry):
        pltpu.make_async_copy(rows_v, z_acc.at[dst_v.at[0]], ssem).wait()
        return carry

    lax.fori_loop(0, AHEAD, drain, 0)


def _deg_pass_body(dst_hbm, z_out, dst_v, rows_v, zbuf, z_acc, ssem):
    cid = lax.axis_index("c")
    sid = lax.axis_index("s")
    rows = pl.ds(sid * ROWS_PT, ROWS_PT)
    _zero_acc(zbuf, z_acc, rows)

    @pl.when(cid == 0)
    def _stage0():
        pltpu.sync_copy(dst_hbm.at[pl.ds(sid * GSUM, D0)],
                        dst_v.at[pl.ds(0, D0)])

    @pl.when(cid == 1)
    def _stage1():
        pltpu.sync_copy(dst_hbm.at[pl.ds(sid * GSUM + D0, D1)],
                        dst_v.at[pl.ds(0, D1)])

    one = jnp.where(lax.iota(jnp.int32, 16) == 0, 1.0, 0.0).astype(jnp.float32)
    zv = jnp.zeros((16,), jnp.float32)

    def fill(r, carry):
        rows_v[r, pl.ds(0, 16)] = one
        rows_v[r, pl.ds(16, 16)] = zv
        return carry

    lax.fori_loop(0, GROUP, fill, 0)
    plsc.subcore_barrier()

    @pl.when(cid == 0)
    def _run0():
        _deg_fire(z_acc, dst_v, rows_v, ssem, D0)

    @pl.when(cid == 1)
    def _run1():
        _deg_fire(z_acc, dst_v, rows_v, ssem, D1)

    plsc.subcore_barrier()
    pltpu.sync_copy(z_acc.at[rows], z_out.at[cid].at[rows])


@functools.cache
def _edge_pass():
    return pl.kernel(
        _edge_pass_body,
        out_type=jax.ShapeDtypeStruct((NC, N_PAD, H), jnp.float32),
        mesh=plsc.VectorSubcoreMesh(core_axis_name="c", subcore_axis_name="s",
                                    num_cores=NC, num_subcores=NS),
        scratch_types=[
            pltpu.VMEM((G0, GROUP), jnp.int32),
            pltpu.VMEM((G0, GROUP), jnp.int32),
            pltpu.VMEM((NBUF, GROUP, H), jnp.float32),
            pltpu.VMEM((ROWS_PT, H), jnp.float32),
            pltpu.VMEM_SHARED((N_PAD, H), jnp.float32),
            pltpu.VMEM_SHARED((N_PAD, H), jnp.float32),
            pltpu.SemaphoreType.DMA((NBUF,)),
            pltpu.SemaphoreType.DMA((NBUF,)),
        ],
        compiler_params=pltpu.CompilerParams(use_tc_tiling_on_sc=False),
    )


@functools.cache
def _deg_pass():
    return pl.kernel(
        _deg_pass_body,
        out_type=jax.ShapeDtypeStruct((NC, N_PAD, H), jnp.float32),
        mesh=plsc.VectorSubcoreMesh(core_axis_name="c", subcore_axis_name="s",
                                    num_cores=NC, num_subcores=NS),
        scratch_types=[
            pltpu.VMEM((D0, GROUP), jnp.int32),
            pltpu.VMEM((GROUP, H), jnp.float32),
            pltpu.VMEM((ROWS_PT, H), jnp.float32),
            pltpu.VMEM_SHARED((N_PAD, H), jnp.float32),
            pltpu.SemaphoreType.DMA,
        ],
        compiler_params=pltpu.CompilerParams(use_tc_tiling_on_sc=False),
    )


_PREC = jax.lax.Precision.HIGHEST


def _tc1_body(x_ref, w1_ref, degp_ref, xw_ref, y_ref, dinv_ref):
    deg = (jnp.sum(degp_ref[...], axis=(0, 2))[:N_NODES] + 1.0)
    dinv = lax.rsqrt(deg)
    xw = jnp.dot(x_ref[...], w1_ref[...], precision=_PREC,
                 preferred_element_type=jnp.float32)
    xw_ref[...] = xw
    y_ref[...] = jnp.zeros((N_PAD, H), jnp.float32)
    y_ref[pl.ds(0, N_NODES)] = xw * dinv[:, None]
    dinv_ref[...] = dinv


def _tc2_body(zp_ref, xw_ref, dinv_ref, b_ref, w_ref, xw2_ref, y2_ref):
    z = zp_ref[0, :N_NODES, :] + zp_ref[1, :N_NODES, :]
    dinv = dinv_ref[...]
    h = jnp.maximum(dinv[:, None] * z + (dinv * dinv)[:, None] * xw_ref[...]
                    + b_ref[...][None, :], 0.0)
    xw2 = jnp.dot(h, w_ref[...], precision=_PREC,
                  preferred_element_type=jnp.float32)
    xw2_ref[...] = xw2
    y2_ref[...] = jnp.zeros((N_PAD, H), jnp.float32)
    y2_ref[pl.ds(0, N_NODES)] = xw2 * dinv[:, None]


def _tc3_body(zp_ref, xw_ref, dinv_ref, b_ref, wh_ref, bh_ref, cash_ref,
              w0_ref, wr_ref):
    z = zp_ref[0, :N_NODES, :] + zp_ref[1, :N_NODES, :]
    dinv = dinv_ref[...]
    h = jnp.maximum(dinv[:, None] * z + (dinv * dinv)[:, None] * xw_ref[...]
                    + b_ref[...][None, :], 0.0)
    s = jnp.dot(h, wh_ref[...], precision=_PREC,
                preferred_element_type=jnp.float32)[:, 0] + bh_ref[...]
    m = jnp.maximum(jnp.max(s), jnp.max(cash_ref[...]))
    es = jnp.exp(s - m)
    ec = jnp.exp(cash_ref[...] - m)
    tot = jnp.sum(es) + jnp.sum(ec)
    w0_ref[...] = ec / tot
    wr_ref[...] = es / tot


def kernel(x, edge_index, W1, b1, W2, b2, Wh, bh, cash):
    ei = edge_index.astype(jnp.int32)
    pad = E_PAD - E
    src = jnp.concatenate([ei[0], jnp.zeros((pad,), jnp.int32)])
    dst = jnp.concatenate([ei[1], jnp.full((pad,), N_NODES, jnp.int32)])
    src = src.reshape(NW * GPT, GROUP)
    dst = dst.reshape(NW * GPT, GROUP)
    degp = _deg_pass()(dst)
    xw1, y1, dinv = pl.pallas_call(
        _tc1_body,
        out_shape=[
            jax.ShapeDtypeStruct((N_NODES, H), jnp.float32),
            jax.ShapeDtypeStruct((N_PAD, H), jnp.float32),
            jax.ShapeDtypeStruct((N_NODES,), jnp.float32),
        ],
    )(x, W1, degp)
    z1p = _edge_pass()(src, dst, y1)
    xw2, y2 = pl.pallas_call(
        _tc2_body,
        out_shape=[
            jax.ShapeDtypeStruct((N_NODES, H), jnp.float32),
            jax.ShapeDtypeStruct((N_PAD, H), jnp.float32),
        ],
    )(z1p, xw1, dinv, b1, W2)
    z2p = _edge_pass()(src, dst, y2)
    w0, wr = pl.pallas_call(
        _tc3_body,
        out_shape=[
            jax.ShapeDtypeStruct((1,), jnp.float32),
            jax.ShapeDtypeStruct((N_NODES,), jnp.float32),
        ],
    )(z2p, xw2, dinv, b2, Wh, bh, cash)
    return jnp.concatenate([w0, wr], axis=0)


# speedup vs baseline: 1.6178x; 1.0206x over previous
import functools

import jax
import jax.numpy as jnp
from jax import lax
from jax.experimental import pallas as pl
from jax.experimental.pallas import tpu as pltpu
from jax.experimental.pallas import tpu_sc as plsc

N_NODES = 10000
N_PAD = 10112
E = 320000
IN_CH = 128
H = 32
DW = 16
NC, NS = 2, 16
NW = NC * NS
GROUP = 256
GPT = 40
E_PAD = NW * GPT * GROUP
ROWS_PT = N_PAD // NS
LAST_PT = N_NODES - (NS - 1) * ROWS_PT
NBUF = 4
G0, G1 = 40, 40
D0, D1 = 40, 40
GSUM = G0 + G1


def _pipe(y_sp, z_acc, src_v, dst_v, rows4, gsem, ssem, G):
    NP = G // NBUF
    for b in range(NBUF):
        pltpu.async_copy(y_sp.at[src_v.at[b]], rows4.at[b], gsem.at[b])

    def step(P, carry):
        for b in range(NBUF):
            g = P * NBUF + b
            pltpu.make_async_copy(y_sp.at[src_v.at[g]], rows4.at[b],
                                  gsem.at[b]).wait()
            pltpu.async_copy(rows4.at[b], z_acc.at[dst_v.at[g]], ssem.at[b],
                             add=True)

            @pl.when(P < NP - 1)
            def _refill():
                pltpu.make_async_copy(rows4.at[b], z_acc.at[dst_v.at[g]],
                                      ssem.at[b]).wait()
                pltpu.async_copy(y_sp.at[src_v.at[g + NBUF]], rows4.at[b],
                                 gsem.at[b])
        return carry

    lax.fori_loop(0, NP, step, 0)
    for b in range(NBUF):
        g = (NP - 1) * NBUF + b
        pltpu.make_async_copy(rows4.at[b], z_acc.at[dst_v.at[g]],
                              ssem.at[b]).wait()


def _edge_pass_body(src_hbm, dst_hbm, y_hbm, zeros_hbm, z_out,
                    src_v, dst_v, rows4, y_sp, z_acc, gsem, ssem):
    cid = lax.axis_index("c")
    sid = lax.axis_index("s")
    rows = pl.ds(sid * ROWS_PT, ROWS_PT)
    pltpu.sync_copy(zeros_hbm.at[rows], z_acc.at[rows])

    @pl.when(sid < NS - 1)
    def _stage_y():
        pltpu.sync_copy(y_hbm.at[rows], y_sp.at[rows])

    @pl.when(sid == NS - 1)
    def _stage_y_last():
        tail = pl.ds((NS - 1) * ROWS_PT, LAST_PT)
        pltpu.sync_copy(y_hbm.at[tail], y_sp.at[tail])

    @pl.when(cid == 0)
    def _stage0():
        base = sid * GSUM
        pltpu.sync_copy(src_hbm.at[pl.ds(base, G0)], src_v.at[pl.ds(0, G0)])
        pltpu.sync_copy(dst_hbm.at[pl.ds(base, G0)], dst_v.at[pl.ds(0, G0)])

    @pl.when(cid == 1)
    def _stage1():
        base = sid * GSUM + G0
        pltpu.sync_copy(src_hbm.at[pl.ds(base, G1)], src_v.at[pl.ds(0, G1)])
        pltpu.sync_copy(dst_hbm.at[pl.ds(base, G1)], dst_v.at[pl.ds(0, G1)])

    plsc.subcore_barrier()

    @pl.when(cid == 0)
    def _run0():
        _pipe(y_sp, z_acc, src_v, dst_v, rows4, gsem, ssem, G0)

    @pl.when(cid == 1)
    def _run1():
        _pipe(y_sp, z_acc, src_v, dst_v, rows4, gsem, ssem, G1)

    plsc.subcore_barrier()

    @pl.when(sid < NS - 1)
    def _pub():
        pltpu.sync_copy(z_acc.at[rows], z_out.at[cid].at[rows])

    @pl.when(sid == NS - 1)
    def _pub_last():
        tail = pl.ds((NS - 1) * ROWS_PT, LAST_PT)
        pltpu.sync_copy(z_acc.at[tail], z_out.at[cid].at[tail])


def _deg_fire(z_acc, dst_v, rows_v, ssem, G):
    AHEAD = 8

    def fire(g, carry):
        pltpu.async_copy(rows_v, z_acc.at[dst_v.at[g]], ssem, add=True)

        @pl.when(g >= AHEAD)
        def _drain():
            pltpu.make_async_copy(rows_v, z_acc.at[dst_v.at[g]], ssem).wait()
        return carry

    lax.fori_loop(0, G, fire, 0)

    def drain(g, carry):
        pltpu.make_async_copy(rows_v, z_acc.at[dst_v.at[0]], ssem).wait()
        return carry

    lax.fori_loop(0, AHEAD, drain, 0)


def _deg_pass_body(dst_hbm, zeros_hbm, e0_hbm, z_out,
                   dst_v, rows_v, z_acc, ssem):
    cid = lax.axis_index("c")
    sid = lax.axis_index("s")
    rows = pl.ds(sid * ROWS_PT, ROWS_PT)
    pltpu.sync_copy(zeros_hbm.at[rows], z_acc.at[rows])
    pltpu.sync_copy(e0_hbm, rows_v)

    @pl.when(cid == 0)
    def _stage0():
        pltpu.sync_copy(dst_hbm.at[pl.ds(sid * GSUM, D0)],
                        dst_v.at[pl.ds(0, D0)])

    @pl.when(cid == 1)
    def _stage1():
        pltpu.sync_copy(dst_hbm.at[pl.ds(sid * GSUM + D0, D1)],
                        dst_v.at[pl.ds(0, D1)])

    plsc.subcore_barrier()

    @pl.when(cid == 0)
    def _run0():
        _deg_fire(z_acc, dst_v, rows_v, ssem, D0)

    @pl.when(cid == 1)
    def _run1():
        _deg_fire(z_acc, dst_v, rows_v, ssem, D1)

    plsc.subcore_barrier()

    @pl.when(sid < NS - 1)
    def _pub():
        pltpu.sync_copy(z_acc.at[rows], z_out.at[cid].at[rows])

    @pl.when(sid == NS - 1)
    def _pub_last():
        tail = pl.ds((NS - 1) * ROWS_PT, LAST_PT)
        pltpu.sync_copy(z_acc.at[tail], z_out.at[cid].at[tail])


@functools.cache
def _edge_pass():
    return pl.kernel(
        _edge_pass_body,
        out_type=jax.ShapeDtypeStruct((NC, N_NODES, H), jnp.float32),
        mesh=plsc.VectorSubcoreMesh(core_axis_name="c", subcore_axis_name="s",
                                    num_cores=NC, num_subcores=NS),
        scratch_types=[
            pltpu.VMEM((max(G0, G1), GROUP), jnp.int32),
            pltpu.VMEM((max(G0, G1), GROUP), jnp.int32),
            pltpu.VMEM((NBUF, GROUP, H), jnp.float32),
            pltpu.VMEM_SHARED((N_PAD, H), jnp.float32),
            pltpu.VMEM_SHARED((N_PAD, H), jnp.float32),
            pltpu.SemaphoreType.DMA((NBUF,)),
            pltpu.SemaphoreType.DMA((NBUF,)),
        ],
        compiler_params=pltpu.CompilerParams(use_tc_tiling_on_sc=False),
    )


@functools.cache
def _deg_pass():
    return pl.kernel(
        _deg_pass_body,
        out_type=jax.ShapeDtypeStruct((NC, N_NODES, DW), jnp.float32),
        mesh=plsc.VectorSubcoreMesh(core_axis_name="c", subcore_axis_name="s",
                                    num_cores=NC, num_subcores=NS),
        scratch_types=[
            pltpu.VMEM((max(D0, D1), GROUP), jnp.int32),
            pltpu.VMEM((GROUP, DW), jnp.float32),
            pltpu.VMEM_SHARED((N_PAD, DW), jnp.float32),
            pltpu.SemaphoreType.DMA,
        ],
        compiler_params=pltpu.CompilerParams(use_tc_tiling_on_sc=False),
    )


_PREC = jax.lax.Precision.HIGHEST
BM = 1000


def _tca_body(x_ref, w1_ref, xw_ref):
    xw_ref[...] = jnp.dot(x_ref[...], w1_ref[...], precision=_PREC,
                          preferred_element_type=jnp.float32)


def _tcb_body(degp_ref, xw_ref, y_ref, dinv_ref):
    deg = jnp.sum(degp_ref[...], axis=(0, 2)) + 1.0
    dinv = lax.rsqrt(deg)
    y_ref[...] = xw_ref[...] * dinv[:, None]
    dinv_ref[...] = dinv[:, None]


def _tc2_body(zp_ref, xw_ref, dinv_ref, b_ref, w_ref, xw2_ref, y2_ref):
    z = zp_ref[0] + zp_ref[1]
    dinv = dinv_ref[...][:, 0]
    h = jnp.maximum(dinv[:, None] * z + (dinv * dinv)[:, None] * xw_ref[...]
                    + b_ref[...][None, :], 0.0)
    xw2 = jnp.dot(h, w_ref[...], precision=_PREC,
                  preferred_element_type=jnp.float32)
    xw2_ref[...] = xw2
    y2_ref[...] = xw2 * dinv[:, None]


def _tc3a_body(zp_ref, xw_ref, dinv_ref, b_ref, wh_ref, bh_ref, s_ref):
    z = zp_ref[0] + zp_ref[1]
    dinv = dinv_ref[...][:, 0]
    h = jnp.maximum(dinv[:, None] * z + (dinv * dinv)[:, None] * xw_ref[...]
                    + b_ref[...][None, :], 0.0)
    s_ref[...] = (jnp.dot(h, wh_ref[...], precision=_PREC,
                          preferred_element_type=jnp.float32)
                  + bh_ref[...][None, :])


def _tc3b_body(s_ref, cash_ref, w0_ref, wr_ref):
    s = s_ref[...][:, 0]
    m = jnp.maximum(jnp.max(s), jnp.max(cash_ref[...]))
    es = jnp.exp(s - m)
    ec = jnp.exp(cash_ref[...] - m)
    tot = jnp.sum(es) + jnp.sum(ec)
    w0_ref[...] = ec / tot
    wr_ref[...] = es / tot


def _row_spec(w):
    return pl.BlockSpec((BM, w), lambda i: (i, 0))


def _vec_spec():
    return pl.BlockSpec((BM, 1), lambda i: (i, 0))


def _full_spec(shape):
    nd = len(shape)
    return pl.BlockSpec(shape, lambda i, _nd=nd: (0,) * _nd)


def _part_spec(w):
    return pl.BlockSpec((NC, BM, w), lambda i: (0, i, 0))


_GRID = N_NODES // BM
_f32 = jnp.float32


def kernel(x, edge_index, W1, b1, W2, b2, Wh, bh, cash):
    ei = edge_index.astype(jnp.int32)
    pad = E_PAD - E
    src = jnp.concatenate([ei[0], jnp.zeros((pad,), jnp.int32)])
    dst = jnp.concatenate([ei[1], jnp.full((pad,), N_NODES, jnp.int32)])
    src = src.reshape(NW * GPT, GROUP)
    dst = dst.reshape(NW * GPT, GROUP)
    zeros32 = jnp.zeros((N_PAD, H), _f32)
    zeros16 = jnp.zeros((N_PAD, DW), _f32)
    e0 = jnp.zeros((GROUP, DW), _f32).at[:, 0].set(1.0)

    xw1 = pl.pallas_call(
        _tca_body,
        grid=(_GRID,),
        in_specs=[_row_spec(IN_CH), _full_spec((IN_CH, H))],
        out_specs=_row_spec(H),
        out_shape=jax.ShapeDtypeStruct((N_NODES, H), _f32),
    )(x, W1)
    degp = _deg_pass()(dst, zeros16, e0)
    y1, dinv = pl.pallas_call(
        _tcb_body,
        grid=(_GRID,),
        in_specs=[_part_spec(DW), _row_spec(H)],
        out_specs=[_row_spec(H), _vec_spec()],
        out_shape=[jax.ShapeDtypeStruct((N_NODES, H), _f32),
                   jax.ShapeDtypeStruct((N_NODES, 1), _f32)],
    )(degp, xw1)
    z1p = _edge_pass()(src, dst, y1, zeros32)
    xw2, y2 = pl.pallas_call(
        _tc2_body,
        grid=(_GRID,),
        in_specs=[_part_spec(H), _row_spec(H), _vec_spec(),
                  _full_spec((H,)), _full_spec((H, H))],
        out_specs=[_row_spec(H), _row_spec(H)],
        out_shape=[jax.ShapeDtypeStruct((N_NODES, H), _f32),
                   jax.ShapeDtypeStruct((N_NODES, H), _f32)],
    )(z1p, xw1, dinv, b1, W2)
    z2p = _edge_pass()(src, dst, y2, zeros32)
    s = pl.pallas_call(
        _tc3a_body,
        grid=(_GRID,),
        in_specs=[_part_spec(H), _row_spec(H), _vec_spec(),
                  _full_spec((H,)), _full_spec((H, 1)), _full_spec((1,))],
        out_specs=_vec_spec(),
        out_shape=jax.ShapeDtypeStruct((N_NODES, 1), _f32),
    )(z2p, xw2, dinv, b2, Wh, bh)
    w0, wr = pl.pallas_call(
        _tc3b_body,
        out_shape=[jax.ShapeDtypeStruct((1,), _f32),
                   jax.ShapeDtypeStruct((N_NODES,), _f32)],
    )(s, cash)
    return jnp.concatenate([w0, wr], axis=0)
